# baseline (speedup 1.0000x reference)
import jax
import jax.numpy as jnp
from jax import lax
from jax.experimental import pallas as pl
from jax.experimental.pallas import tpu as pltpu
from jax.experimental.pallas import tpu_sc as plsc

_HOPS = 4
_S = 16
_V = 8192
_B = 64
_NC = 2
_NS = 16
_NW = _NC * _NS
_BPW = _B // _NW
_PAIR = 2
_L = 16


def _body(y_hbm, idx_hbm, out_hbm, slab0, slab1, idx_v, sem0, sem1):
    slabs = [slab0, slab1]
    sems = [sem0, sem1]
    wid = lax.axis_index("s") * _NC + lax.axis_index("c")
    bs = [wid * _BPW + j for j in range(_BPW)]
    rsel = [jnp.full((_L,), r, jnp.int32) for r in range(_PAIR)]
    for j in range(_BPW):
        pltpu.sync_copy(idx_hbm.at[bs[j]], idx_v.at[j])
        for r in range(_PAIR):
            pltpu.sync_copy(y_hbm.at[bs[j]], slabs[j].at[r])
    pending = [None] * _BPW
    saved = [None] * _BPW
    for t in range(_BPW * _S // _PAIR):
        j = t % _BPW
        p = t // _BPW
        if pending[j] is not None:
            pending[j].wait()
            for siv, sv in saved[j]:
                plsc.store_scatter(slabs[j], siv, sv)
        sv_list = []
        for r in range(_PAIR):
            iv = idx_v[j, _PAIR * p + r]
            v = plsc.load_gather(slabs[j], [rsel[r], iv])
            plsc.store_scatter(slabs[j], [rsel[r], iv], 1.0 - v)
            sv_list.append(([rsel[r], iv], v))
        cp = pltpu.make_async_copy(
            slabs[j], out_hbm.at[bs[j], pl.ds(_PAIR * p, _PAIR)], sems[j])
        cp.start()
        pending[j] = cp
        saved[j] = sv_list
    for j in range(_BPW):
        pending[j].wait()


_mesh = plsc.VectorSubcoreMesh(
    core_axis_name="c", subcore_axis_name="s",
    num_cores=_NC, num_subcores=_NS)

_sampler = pl.kernel(
    _body,
    out_type=jax.ShapeDtypeStruct((_B, _S, _V), jnp.float32),
    mesh=_mesh,
    compiler_params=pltpu.CompilerParams(needs_layout_passes=False),
    scratch_types=[
        pltpu.VMEM((_PAIR, _V), jnp.float32),
        pltpu.VMEM((_PAIR, _V), jnp.float32),
        pltpu.VMEM((_BPW, _S, _L), jnp.int32),
        pltpu.SemaphoreType.DMA,
        pltpu.SemaphoreType.DMA,
    ],
)


def kernel(a, b, c, y):
    del a, b, c
    idx = jax.random.randint(jax.random.key(1), (_B, _S, _HOPS), 0, _V)
    idx16 = jnp.tile(idx.astype(jnp.int32), (1, 1, _L // _HOPS))
    return _sampler(y, idx16)


# speedup vs baseline: 1.1164x; 1.1164x over previous
import jax
import jax.numpy as jnp
from jax import lax
from jax.experimental import pallas as pl
from jax.experimental.pallas import tpu as pltpu
from jax.experimental.pallas import tpu_sc as plsc

_HOPS = 4
_S = 16
_V = 8192
_B = 64
_NC = 2
_NS = 16
_NW = _NC * _NS
_BPW = _B // _NW
_NBUF = _BPW
_L = 16


def _body(y_hbm, idx_hbm, out_hbm, *scratch):
    rows = list(scratch[:_NBUF])
    idx_v = scratch[_NBUF]
    sems = list(scratch[_NBUF + 1:])
    wid = lax.axis_index("s") * _NC + lax.axis_index("c")
    bs = [wid * _BPW + j for j in range(_BPW)]
    stage = []
    for j in range(_BPW):
        cp = pltpu.make_async_copy(idx_hbm.at[bs[j]], idx_v.at[j], sems[j])
        cp.start()
        stage.append(cp)
    for i in range(_NBUF):
        cp = pltpu.make_async_copy(y_hbm.at[bs[i % _BPW]], rows[i], sems[i])
        cp.start()
        stage.append(cp)
    for cp in stage:
        cp.wait()
    pending = [None] * _NBUF
    saved = [None] * _NBUF
    for t in range(_BPW * _S):
        i = t % _NBUF
        j = t % _BPW
        s = t // _BPW
        if pending[t % _NBUF] is not None:
            pending[i].wait()
            siv, sv = saved[i]
            plsc.store_scatter(rows[i], [siv], sv)
        iv = idx_v[j, s]
        v = plsc.load_gather(rows[i], [iv])
        plsc.store_scatter(rows[i], [iv], 1.0 - v)
        cp = pltpu.make_async_copy(rows[i], out_hbm.at[bs[j], s], sems[i])
        cp.start()
        pending[i] = cp
        saved[i] = (iv, v)
    for i in range(_NBUF):
        pending[i].wait()


_mesh = plsc.VectorSubcoreMesh(
    core_axis_name="c", subcore_axis_name="s",
    num_cores=_NC, num_subcores=_NS)

_sampler = pl.kernel(
    _body,
    out_type=jax.ShapeDtypeStruct((_B, _S, _V), jnp.float32),
    mesh=_mesh,
    compiler_params=pltpu.CompilerParams(needs_layout_passes=False),
    scratch_types=(
        [pltpu.VMEM((_V,), jnp.float32)] * _NBUF
        + [pltpu.VMEM((_BPW, _S, _L), jnp.int32)]
        + [pltpu.SemaphoreType.DMA] * _NBUF
    ),
)


def kernel(a, b, c, y):
    del a, b, c
    idx = jax.random.randint(jax.random.key(1), (_B, _S, _HOPS), 0, _V)
    idx16 = jnp.tile(idx.astype(jnp.int32), (1, 1, _L // _HOPS))
    return _sampler(y, idx16)
